# Initial kernel scaffold; baseline (speedup 1.0000x reference)
import jax
import jax.numpy as jnp
from jax.experimental import pallas as pl


def kernel(context_output, object_output, eval_random):
    raise NotImplementedError("write your pallas kernel here")


# speedup vs baseline: 1.2044x; 1.2044x over previous
import functools

import jax
import jax.numpy as jnp
from jax import lax
from jax.experimental import pallas as pl
from jax.experimental.pallas import tpu as pltpu
from jax.experimental.pallas import tpu_sc as plsc

N = 100000
D = 128
NW = 32
C = 400
NCHUNK = N // C
ITERS = (NCHUNK + NW - 1) // NW

_mesh = plsc.VectorSubcoreMesh(core_axis_name="c", subcore_axis_name="s")


@functools.partial(
    pl.kernel,
    out_type=jax.ShapeDtypeStruct((N, 2 * D), jnp.float32),
    mesh=_mesh,
    scratch_types=[
        pltpu.VMEM((C,), jnp.int32),
        pltpu.VMEM((C, D), jnp.float32),
        pltpu.VMEM((C, D), jnp.float32),
        pltpu.SemaphoreType.DMA,
    ],
)
def _sc_gather_concat(ctx_hbm, obj_hbm, idx_hbm, out_hbm,
                      idx_v, ctx_v, obj_v, sem):
    wid = lax.axis_index("s") * 2 + lax.axis_index("c")
    for i in range(ITERS):
        chunk = wid + i * NW

        @pl.when(chunk < NCHUNK)
        def _():
            base = chunk * C
            pltpu.sync_copy(idx_hbm.at[pl.ds(base, C)], idx_v)
            pltpu.async_copy(ctx_hbm.at[idx_v], ctx_v, sem).wait()
            pltpu.sync_copy(ctx_v, out_hbm.at[pl.ds(base, C), pl.ds(0, D)])
            pltpu.sync_copy(obj_hbm.at[pl.ds(base, C)], obj_v)
            pltpu.sync_copy(obj_v, out_hbm.at[pl.ds(base, C), pl.ds(D, D)])


def kernel(context_output, object_output, eval_random):
    num = context_output.shape[0]
    perm_idx = jax.random.permutation(jax.random.key(42), num)
    identity_idx = jnp.arange(num)
    random_idx = jnp.where(eval_random, perm_idx, identity_idx).astype(jnp.int32)
    return _sc_gather_concat(context_output, object_output, random_idx)
